# baseline (speedup 1.0000x reference)
import functools

import jax
import jax.numpy as jnp
from jax import lax
from jax.experimental import pallas as pl
from jax.experimental.pallas import tpu as pltpu, tpu_sc as plsc

PATCH = 32
NUM_EXPERTS = 16
DIM = 1024
ALPHA = 0.001
NEG_INF = float("-inf")
HIGHEST = jax.lax.Precision.HIGHEST


def _pool_kernel(pan_ref, ms_ref, out_ref):
    n = 256 // PATCH
    q = jax.lax.broadcasted_iota(jnp.int32, (n * PATCH, PATCH), 0)
    j = jax.lax.broadcasted_iota(jnp.int32, (n * PATCH, PATCH), 1)
    fold = (q % PATCH == j).astype(jnp.float32)
    scale = 1.0 / (n * n)
    for c in range(5):
        xr = pan_ref[0, 0] if c == 0 else ms_ref[0, c - 1]
        xb = xr.astype(jnp.bfloat16).astype(jnp.float32)
        s1 = xb[0:PATCH, :]
        for gh in range(1, n):
            s1 = s1 + xb[gh * PATCH:(gh + 1) * PATCH, :]
        s2 = jnp.dot(s1, fold, preferred_element_type=jnp.float32,
                     precision=HIGHEST)
        out_ref[0, c] = s2 * scale


def _router_kernel(pooled_ref, wp_ref, ws_ref, h_ref, logits_ref):
    pooled = pooled_ref[...]
    wpb = wp_ref[...].astype(jnp.bfloat16).astype(jnp.float32)
    h = jnp.dot(pooled, wpb, preferred_element_type=jnp.float32,
                precision=HIGHEST)
    h_ref[...] = h
    logits_ref[...] = jnp.dot(
        h.astype(jnp.bfloat16), ws_ref[...].astype(jnp.bfloat16),
        preferred_element_type=jnp.float32)


def _sc_allreduce(x, iota, op):
    for sh in (1, 2, 4, 8):
        perm = jnp.bitwise_xor(iota, sh)
        xp = x.at[perm].get(mode="promise_in_bounds")
        x = op(x, xp)
    return x


def _sc_topk_row(lg, iota, k):
    idx_lanes = jnp.zeros((16,), jnp.int32)
    counts = jnp.zeros((16,), jnp.float32)
    vals = lg
    for slot in range(k):
        mvec = _sc_allreduce(vals, iota, jnp.maximum)
        at_max = vals >= mvec
        cand = jnp.where(at_max, iota, NUM_EXPERTS)
        idx = _sc_allreduce(cand, iota, jnp.minimum)
        sel = iota == idx
        idx_lanes = jnp.where(iota == slot, idx, idx_lanes)
        counts = counts + jnp.where(sel, 1.0, 0.0)
        vals = jnp.where(sel, NEG_INF, vals)
    return idx_lanes, counts


def _make_sc_route(B, E, k):
    rows_per = B // 16
    mesh = plsc.VectorSubcoreMesh(
        core_axis_name="c", subcore_axis_name="s", num_cores=1)

    @functools.partial(
        pl.kernel,
        mesh=mesh,
        out_type=[
            jax.ShapeDtypeStruct((B, 16), jnp.int32),
            jax.ShapeDtypeStruct((16,), jnp.float32),
            jax.ShapeDtypeStruct((16, 2, 16), jnp.float32),
        ],
        scratch_types=[
            pltpu.VMEM((rows_per, 16), jnp.float32),
            pltpu.VMEM((rows_per, 16), jnp.int32),
            pltpu.VMEM((2, 16), jnp.float32),
            pltpu.VMEM((16, 2, 16), jnp.float32),
            pltpu.VMEM((16,), jnp.float32),
        ],
    )
    def sc_route(logits_hbm, idx_hbm, aux_hbm, stage_hbm,
                 rows_v, idx_v, part_v, all_v, vec_v):
        sid = lax.axis_index("s")
        iota = lax.broadcasted_iota(jnp.int32, (16,), 0)
        base = sid * rows_per

        pltpu.sync_copy(logits_hbm.at[pl.ds(base, rows_per)], rows_v)

        pi = jnp.zeros((16,), jnp.float32)
        cnt = jnp.zeros((16,), jnp.float32)
        for r in range(rows_per):
            lg = rows_v[r]
            idx_lanes, counts = _sc_topk_row(lg, iota, k)
            idx_v[r] = idx_lanes
            mvec = _sc_allreduce(lg, iota, jnp.maximum)
            ex = jnp.exp(lg - mvec)
            svec = _sc_allreduce(ex, iota, jnp.add)
            pi = pi + ex / svec
            cnt = cnt + counts
        part_v[0] = pi
        part_v[1] = cnt

        pltpu.sync_copy(idx_v, idx_hbm.at[pl.ds(base, rows_per)])

        pltpu.sync_copy(part_v, stage_hbm.at[sid])
        plsc.subcore_barrier()

        @pl.when(sid == 0)
        def _finish():
            pisum = jnp.zeros((16,), jnp.float32)
            cntsum = jnp.zeros((16,), jnp.float32)
            for t in range(16):
                pltpu.sync_copy(stage_hbm.at[t], all_v.at[t])
                pisum = pisum + all_v[t, 0]
                cntsum = cntsum + all_v[t, 1]
            scale = ALPHA * E / (B * float(B * k))
            vec_v[...] = _sc_allreduce(pisum * cntsum, iota, jnp.add) * scale
            pltpu.sync_copy(vec_v, aux_hbm)

    return sc_route


def _expert_kernel(h_ref, idx_ref, wg_ref, wu_ref, wd_ref, out_ref):
    e = pl.program_id(0)
    hb = h_ref[...].astype(jnp.bfloat16)
    g = jnp.dot(hb, wg_ref[0].astype(jnp.bfloat16),
                preferred_element_type=jnp.float32)
    u = jnp.dot(hb, wu_ref[0].astype(jnp.bfloat16),
                preferred_element_type=jnp.float32)
    a = (g * jax.nn.sigmoid(g)) * u
    dn = jnp.dot(a.astype(jnp.bfloat16), wd_ref[0].astype(jnp.bfloat16),
                 preferred_element_type=jnp.float32)
    oe = jnp.maximum(dn, 0.0)
    k = out_ref.shape[1]
    mask = (idx_ref[:, 0:k] == e).astype(jnp.float32)
    contrib = oe[:, None, :] * mask[:, :, None]

    @pl.when(e == 0)
    def _init():
        out_ref[...] = contrib

    @pl.when(e > 0)
    def _acc():
        out_ref[...] += contrib


@jax.jit
def kernel(pan, ms, Wp, Ws, Wg, Wu, Wd):
    B = pan.shape[0]
    k = ms.shape[1]
    C = 1 + ms.shape[1]
    E = Ws.shape[1]

    pooled = pl.pallas_call(
        _pool_kernel,
        grid=(B,),
        in_specs=[
            pl.BlockSpec((1, 1, 256, 256), lambda b: (b, 0, 0, 0)),
            pl.BlockSpec((1, 4, 256, 256), lambda b: (b, 0, 0, 0)),
        ],
        out_specs=pl.BlockSpec((1, C, PATCH, PATCH), lambda b: (b, 0, 0, 0)),
        out_shape=jax.ShapeDtypeStruct((B, C, PATCH, PATCH), jnp.float32),
    )(pan, ms)
    pooled = pooled.reshape(B, C * PATCH * PATCH)

    h, logits = pl.pallas_call(
        _router_kernel,
        in_specs=[
            pl.BlockSpec(pooled.shape, lambda: (0, 0)),
            pl.BlockSpec(Wp.shape, lambda: (0, 0)),
            pl.BlockSpec(Ws.shape, lambda: (0, 0)),
        ],
        out_specs=[
            pl.BlockSpec((B, DIM), lambda: (0, 0)),
            pl.BlockSpec((B, E), lambda: (0, 0)),
        ],
        out_shape=[
            jax.ShapeDtypeStruct((B, DIM), jnp.float32),
            jax.ShapeDtypeStruct((B, E), jnp.float32),
        ],
    )(pooled, Wp, Ws)

    topk_idx, auxv, _ = _make_sc_route(B, E, k)(logits)

    selected = pl.pallas_call(
        _expert_kernel,
        grid=(E,),
        in_specs=[
            pl.BlockSpec((B, DIM), lambda e: (0, 0)),
            pl.BlockSpec((B, E), lambda e: (0, 0)),
            pl.BlockSpec((1, DIM, DIM), lambda e: (e, 0, 0)),
            pl.BlockSpec((1, DIM, DIM), lambda e: (e, 0, 0)),
            pl.BlockSpec((1, DIM, DIM), lambda e: (e, 0, 0)),
        ],
        out_specs=pl.BlockSpec((B, k, DIM), lambda e: (0, 0, 0)),
        out_shape=jax.ShapeDtypeStruct((B, k, DIM), jnp.float32),
    )(h, topk_idx, Wg, Wu, Wd)

    return selected, auxv[0]


# speedup vs baseline: 1.0479x; 1.0479x over previous
import functools

import jax
import jax.numpy as jnp
from jax import lax
from jax.experimental import pallas as pl
from jax.experimental.pallas import tpu as pltpu, tpu_sc as plsc

PATCH = 32
NUM_EXPERTS = 16
DIM = 1024
ALPHA = 0.001
NEG_INF = float("-inf")
HIGHEST = jax.lax.Precision.HIGHEST


def _pool_kernel(pan_ref, ms_ref, out_ref):
    n = 256 // PATCH
    q = jax.lax.broadcasted_iota(jnp.int32, (n * PATCH, PATCH), 0)
    j = jax.lax.broadcasted_iota(jnp.int32, (n * PATCH, PATCH), 1)
    fold = (q % PATCH == j).astype(jnp.float32)
    scale = 1.0 / (n * n)
    for c in range(5):
        xr = pan_ref[0, 0] if c == 0 else ms_ref[0, c - 1]
        xb = xr.astype(jnp.bfloat16).astype(jnp.float32)
        s1 = xb[0:PATCH, :]
        for gh in range(1, n):
            s1 = s1 + xb[gh * PATCH:(gh + 1) * PATCH, :]
        s2 = jnp.dot(s1, fold, preferred_element_type=jnp.float32,
                     precision=HIGHEST)
        out_ref[0, c] = s2 * scale


def _router_kernel(pooled_ref, wp_ref, ws_ref, h_ref, logits_ref):
    pooled = pooled_ref[...]
    wpb = wp_ref[...].astype(jnp.bfloat16).astype(jnp.float32)
    h = jnp.dot(pooled, wpb, preferred_element_type=jnp.float32,
                precision=HIGHEST)
    h_ref[...] = h
    logits_ref[...] = jnp.dot(
        h.astype(jnp.bfloat16), ws_ref[...].astype(jnp.bfloat16),
        preferred_element_type=jnp.float32)


def _sc_allreduce(x, iota, op):
    for sh in (1, 2, 4, 8):
        perm = jnp.bitwise_xor(iota, sh)
        xp = x.at[perm].get(mode="promise_in_bounds")
        x = op(x, xp)
    return x


def _sc_topk_row(lg, iota, k):
    idx_lanes = jnp.zeros((16,), jnp.int32)
    counts = jnp.zeros((16,), jnp.float32)
    vals = lg
    for slot in range(k):
        mvec = _sc_allreduce(vals, iota, jnp.maximum)
        at_max = vals >= mvec
        cand = jnp.where(at_max, iota, NUM_EXPERTS)
        idx = _sc_allreduce(cand, iota, jnp.minimum)
        sel = iota == idx
        idx_lanes = jnp.where(iota == slot, idx, idx_lanes)
        counts = counts + jnp.where(sel, 1.0, 0.0)
        vals = jnp.where(sel, NEG_INF, vals)
    return idx_lanes, counts


def _make_sc_route(B, E, k):
    rows_per = B // 16
    mesh = plsc.VectorSubcoreMesh(
        core_axis_name="c", subcore_axis_name="s", num_cores=1)

    @functools.partial(
        pl.kernel,
        mesh=mesh,
        out_type=[
            jax.ShapeDtypeStruct((B, 16), jnp.int32),
            jax.ShapeDtypeStruct((16,), jnp.float32),
            jax.ShapeDtypeStruct((16, 2, 16), jnp.float32),
        ],
        scratch_types=[
            pltpu.VMEM((rows_per, 16), jnp.float32),
            pltpu.VMEM((rows_per, 16), jnp.int32),
            pltpu.VMEM((2, 16), jnp.float32),
            pltpu.VMEM((16, 2, 16), jnp.float32),
            pltpu.VMEM((16,), jnp.float32),
        ],
    )
    def sc_route(logits_hbm, idx_hbm, aux_hbm, stage_hbm,
                 rows_v, idx_v, part_v, all_v, vec_v):
        sid = lax.axis_index("s")
        iota = lax.broadcasted_iota(jnp.int32, (16,), 0)
        base = sid * rows_per

        pltpu.sync_copy(logits_hbm.at[pl.ds(base, rows_per)], rows_v)

        pi = jnp.zeros((16,), jnp.float32)
        cnt = jnp.zeros((16,), jnp.float32)
        for r in range(rows_per):
            lg = rows_v[r]
            idx_lanes, counts = _sc_topk_row(lg, iota, k)
            idx_v[r] = idx_lanes
            mvec = _sc_allreduce(lg, iota, jnp.maximum)
            ex = jnp.exp(lg - mvec)
            svec = _sc_allreduce(ex, iota, jnp.add)
            pi = pi + ex / svec
            cnt = cnt + counts
        part_v[0] = pi
        part_v[1] = cnt

        pltpu.sync_copy(idx_v, idx_hbm.at[pl.ds(base, rows_per)])

        pltpu.sync_copy(part_v, stage_hbm.at[sid])
        plsc.subcore_barrier()

        @pl.when(sid == 0)
        def _finish():
            pisum = jnp.zeros((16,), jnp.float32)
            cntsum = jnp.zeros((16,), jnp.float32)
            for t in range(16):
                pltpu.sync_copy(stage_hbm.at[t], all_v.at[t])
                pisum = pisum + all_v[t, 0]
                cntsum = cntsum + all_v[t, 1]
            scale = ALPHA * E / (B * float(B * k))
            vec_v[...] = _sc_allreduce(pisum * cntsum, iota, jnp.add) * scale
            pltpu.sync_copy(vec_v, aux_hbm)

    return sc_route


def _expert_kernel(h_ref, wg_ref, wu_ref, wd_ref, out_ref):
    hb = h_ref[...].astype(jnp.bfloat16)
    g = jnp.dot(hb, wg_ref[0].astype(jnp.bfloat16),
                preferred_element_type=jnp.float32)
    u = jnp.dot(hb, wu_ref[0].astype(jnp.bfloat16),
                preferred_element_type=jnp.float32)
    a = (g * jax.nn.sigmoid(g)) * u
    dn = jnp.dot(a.astype(jnp.bfloat16), wd_ref[0].astype(jnp.bfloat16),
                 preferred_element_type=jnp.float32)
    out_ref[0] = jnp.maximum(dn, 0.0)


def _gather_kernel(eo_ref, idx_ref, out_ref):
    B, k, D = out_ref.shape
    E = eo_ref.shape[0]
    acc = jnp.zeros((B, k, D), jnp.float32)
    for e in range(E):
        mask = (idx_ref[:, 0:k] == e).astype(jnp.float32)
        acc = acc + eo_ref[e][:, None, :] * mask[:, :, None]
    out_ref[...] = acc


@jax.jit
def kernel(pan, ms, Wp, Ws, Wg, Wu, Wd):
    B = pan.shape[0]
    k = ms.shape[1]
    C = 1 + ms.shape[1]
    E = Ws.shape[1]

    pooled = pl.pallas_call(
        _pool_kernel,
        grid=(B,),
        in_specs=[
            pl.BlockSpec((1, 1, 256, 256), lambda b: (b, 0, 0, 0)),
            pl.BlockSpec((1, 4, 256, 256), lambda b: (b, 0, 0, 0)),
        ],
        out_specs=pl.BlockSpec((1, C, PATCH, PATCH), lambda b: (b, 0, 0, 0)),
        out_shape=jax.ShapeDtypeStruct((B, C, PATCH, PATCH), jnp.float32),
    )(pan, ms)
    pooled = pooled.reshape(B, C * PATCH * PATCH)

    h, logits = pl.pallas_call(
        _router_kernel,
        in_specs=[
            pl.BlockSpec(pooled.shape, lambda: (0, 0)),
            pl.BlockSpec(Wp.shape, lambda: (0, 0)),
            pl.BlockSpec(Ws.shape, lambda: (0, 0)),
        ],
        out_specs=[
            pl.BlockSpec((B, DIM), lambda: (0, 0)),
            pl.BlockSpec((B, E), lambda: (0, 0)),
        ],
        out_shape=[
            jax.ShapeDtypeStruct((B, DIM), jnp.float32),
            jax.ShapeDtypeStruct((B, E), jnp.float32),
        ],
    )(pooled, Wp, Ws)

    topk_idx, auxv, _ = _make_sc_route(B, E, k)(logits)

    expert_out = pl.pallas_call(
        _expert_kernel,
        grid=(E,),
        in_specs=[
            pl.BlockSpec((B, DIM), lambda e: (0, 0)),
            pl.BlockSpec((1, DIM, DIM), lambda e: (e, 0, 0)),
            pl.BlockSpec((1, DIM, DIM), lambda e: (e, 0, 0)),
            pl.BlockSpec((1, DIM, DIM), lambda e: (e, 0, 0)),
        ],
        out_specs=pl.BlockSpec((1, B, DIM), lambda e: (e, 0, 0)),
        out_shape=jax.ShapeDtypeStruct((E, B, DIM), jnp.float32),
    )(h, Wg, Wu, Wd)

    selected = pl.pallas_call(
        _gather_kernel,
        in_specs=[
            pl.BlockSpec((E, B, DIM), lambda: (0, 0, 0)),
            pl.BlockSpec((B, E), lambda: (0, 0)),
        ],
        out_specs=pl.BlockSpec((B, k, DIM), lambda: (0, 0, 0)),
        out_shape=jax.ShapeDtypeStruct((B, k, DIM), jnp.float32),
    )(expert_out, topk_idx)

    return selected, auxv[0]
